# Initial kernel scaffold; baseline (speedup 1.0000x reference)
import jax
import jax.numpy as jnp
from jax.experimental import pallas as pl


def kernel(unique, mapping, primitives):
    raise NotImplementedError("write your pallas kernel here")


# speedup vs baseline: 3.9262x; 3.9262x over previous
import functools

import jax
import jax.numpy as jnp
from jax import lax
from jax.experimental import pallas as pl
from jax.experimental.pallas import tpu as pltpu
from jax.experimental.pallas import tpu_sc as plsc

U = 1024
DIM = 64
DEPTH = 16

B = 4096 * 200
IDXW = 128
NC, NS = 2, 16
NW = NC * NS
ROWS_PER_W = B // IDXW // NW
CHUNK = 8
NCHUNK = ROWS_PER_W // CHUNK


def _embed_body(unique_ref, prim_ref, out_ref):
    u = unique_ref[:]
    m0 = prim_ref[0]
    m1 = prim_ref[1]
    maps = jnp.ones((U, DIM), jnp.float32)
    dn = (((1,), (1,)), ((), ()))
    for depth in range(DEPTH):
        shifted = u >> depth
        code = jnp.where(shifted > 1, shifted & 1, 2)
        a = lax.dot_general(maps, m0, dn, preferred_element_type=jnp.float32)
        b = lax.dot_general(maps, m1, dn, preferred_element_type=jnp.float32)
        maps = jnp.where(code == 0, a, jnp.where(code == 1, b, maps))
    out_ref[:] = maps


def _embed(unique, primitives):
    return pl.pallas_call(
        _embed_body,
        out_shape=jax.ShapeDtypeStruct((U, DIM), jnp.float32),
    )(unique.reshape(U, 1), primitives)


def _gather_body(table_hbm, idx_hbm, out_hbm, idx_v, rows_v, sem):
    wid = lax.axis_index("s") * NC + lax.axis_index("c")

    def chunk(c, _):
        row0 = wid * ROWS_PER_W + c * CHUNK
        pltpu.sync_copy(idx_hbm.at[pl.ds(row0, CHUNK)], idx_v)
        copies = []
        for j in range(CHUNK):
            copies.append(pltpu.async_copy(
                table_hbm.at[idx_v.at[j]],
                rows_v.at[pl.ds(j * IDXW, IDXW)],
                sem,
            ))
        for cp in copies:
            cp.wait()
        pltpu.sync_copy(rows_v, out_hbm.at[pl.ds(row0 * IDXW, CHUNK * IDXW)])
        return ()

    lax.fori_loop(0, NCHUNK, chunk, (), unroll=False)


@functools.partial(jax.jit, static_argnums=())
def _gather(table, idx2d):
    mesh = plsc.VectorSubcoreMesh(core_axis_name="c", subcore_axis_name="s")
    f = pl.kernel(
        _gather_body,
        out_type=jax.ShapeDtypeStruct((B, DIM), jnp.float32),
        mesh=mesh,
        scratch_types=[
            pltpu.VMEM((CHUNK, IDXW), jnp.int32),
            pltpu.VMEM((CHUNK * IDXW, DIM), jnp.float32),
            pltpu.SemaphoreType.DMA,
        ],
        compiler_params=pltpu.CompilerParams(use_tc_tiling_on_sc=False),
    )
    return f(table, idx2d)


def kernel(unique, mapping, primitives):
    table = _embed(unique, primitives)
    idx2d = mapping.reshape(B // IDXW, IDXW)
    out = _gather(table, idx2d)
    return out.reshape(*mapping.shape, DIM)
